# baseline (speedup 1.0000x reference)
import functools

import jax
import jax.numpy as jnp
from jax.experimental import pallas as pl
from jax.experimental.pallas import tpu as pltpu

_N = 10000
_E = 320000
_DF = 128
_H = 256
_C = 40
_L = 4
_K = 10
_ALPHA = 0.1

_NP = 10240
_RB = 1024
_CP = 128


def _mlp_body(x_ref, w1_ref, b1_ref, w2_ref, b2_ref, wa_ref, ba_ref,
              wb_ref, bb_ref, o_ref, h_ref):
    x = x_ref[...]
    h1 = jnp.maximum(
        jnp.dot(x, w1_ref[...], preferred_element_type=jnp.float32)
        + b1_ref[...], 0.0)
    hl = (jnp.dot(h1, w2_ref[...], preferred_element_type=jnp.float32)
          + b2_ref[...])
    h_ref[...] = hl
    for i in range(_L):
        a = jnp.maximum(
            jnp.dot(hl, wa_ref[i], preferred_element_type=jnp.float32)
            + ba_ref[i], 0.0)
        o_ref[i] = (jnp.dot(a, wb_ref[i], preferred_element_type=jnp.float32)
                    + bb_ref[i])


def _mlp_pallas(x, W1, b1, W2, b2, Wa, ba, Wb, bb):
    nblk = _NP // _RB
    grid = (nblk,)
    full = lambda shape: pl.BlockSpec(shape, lambda i: tuple(0 for _ in shape))
    o, h = pl.pallas_call(
        _mlp_body,
        grid=grid,
        in_specs=[
            pl.BlockSpec((_RB, _DF), lambda i: (i, 0)),
            full((_DF, _H)),
            full((1, _H)),
            full((_H, _H)),
            full((1, _H)),
            full((_L, _H, _H // 2)),
            full((_L, 1, _H // 2)),
            full((_L, _H // 2, _CP)),
            full((_L, 1, _CP)),
        ],
        out_specs=[
            pl.BlockSpec((_L, _RB, _CP), lambda i: (0, i, 0)),
            pl.BlockSpec((_RB, _H), lambda i: (i, 0)),
        ],
        out_shape=[
            jax.ShapeDtypeStruct((_L, _NP, _CP), jnp.float32),
            jax.ShapeDtypeStruct((_NP, _H), jnp.float32),
        ],
    )(x, W1, b1.reshape(1, _H), W2, b2.reshape(1, _H),
      Wa, ba.reshape(_L, 1, _H // 2),
      jnp.pad(Wb, ((0, 0), (0, 0), (0, _CP - _C))),
      jnp.pad(bb, ((0, 0), (0, _CP - _C))).reshape(_L, 1, _CP))
    return o, h


def _appnp_steps(h0, src, dst, norm, k):
    h = h0
    for _ in range(k):
        h = h * norm[:, None]
        m = jax.ops.segment_sum(h[src], dst, num_segments=_N)
        h = m * norm[:, None]
        h = (1.0 - _ALPHA) * h + _ALPHA * h0
    return h


def kernel(input_feat, edge_index, W1, b1, W2, b2, Wa, ba, Wb, bb):
    src = edge_index[0]
    dst = edge_index[1]
    deg = jax.ops.segment_sum(jnp.ones((_E,), dtype=jnp.float32), dst,
                              num_segments=_N)
    norm = jnp.where(deg > 0, jax.lax.rsqrt(jnp.maximum(deg, 1.0)), 0.0)
    h = _appnp_steps(input_feat, src, dst, norm, _K)
    h = _appnp_steps(h, src, dst, norm, _K)

    hp = jnp.pad(h, ((0, _NP - _N), (0, 0)))
    o, h_last = _mlp_pallas(hp, W1, b1, W2, b2, Wa, ba, Wb, bb)
    outs = tuple(o[i, :_N, :_C] for i in range(_L))
    return (*outs, h_last[:_N])


# speedup vs baseline: 4.2298x; 4.2298x over previous
import functools

import jax
import jax.numpy as jnp
from jax import lax
from jax.experimental import pallas as pl
from jax.experimental.pallas import tpu as pltpu
from jax.experimental.pallas import tpu_sc as plsc

_N = 10000
_E = 320000
_DF = 128
_H = 256
_C = 40
_L = 4
_K = 10
_ALPHA = 0.1

_NC = 2
_NS = 16
_NP = 10240
_NT = _NP // _NS
_CH = 128
_NCHUNK = 157
_EP = _NS * _NCHUNK * _CH
_EPT = _NCHUNK * _CH
_FH = _DF // _NC
_RB = 1024
_CP = 128


def _rsqrt16(d):
    i = plsc.bitcast(d, jnp.int32)
    i = jnp.int32(0x5F3759DF) - (i >> 1)
    y = plsc.bitcast(i, jnp.float32)
    for _ in range(3):
        y = y * (1.5 - 0.5 * d * y * y)
    return y


def _prop_body(x_hbm, src_hbm, dst_hbm, out_hbm,
               g_sh, m_sh, sidx, didx, st1, st2, zb, nrm, sem):
    c = lax.axis_index("c")
    s = lax.axis_index("s")
    rowbase = s * _NT
    colbase = c * _FH
    zero16 = jnp.zeros((16,), jnp.float32)
    lane = lax.iota(jnp.int32, 16)
    zlane = jnp.zeros((16,), jnp.int32)
    e0 = jnp.where(lane == 0, 1.0, 0.0).astype(jnp.float32)
    nslice = lambda t: pl.ds(rowbase + t * _CH, _CH)

    def _zrow(r, _):
        for q in range(_FH // 16):
            sl = pl.ds(q * 16, 16)
            zb[r, sl] = zero16
            st1[r, sl] = e0 if q == 0 else zero16
        return 0
    lax.fori_loop(0, _CH, _zrow, 0)

    for t in range(_NT // _CH):
        pltpu.sync_copy(zb, m_sh.at[nslice(t)])
    plsc.subcore_barrier()

    def _degc(j, _):
        pltpu.sync_copy(dst_hbm.at[s, j], didx)
        pltpu.sync_copy(st1, m_sh.at[didx], add=True)
        return 0
    lax.fori_loop(0, _NCHUNK, _degc, 0)
    plsc.subcore_barrier()

    for t in range(_NT // _CH):
        pltpu.sync_copy(m_sh.at[nslice(t)], st2)

        def _ext(i, _):
            d = plsc.load_gather(st2, [i * 16 + lane, zlane])
            y = _rsqrt16(jnp.maximum(d, 1.0))
            nrm[pl.ds(t * _CH + i * 16, 16)] = jnp.where(d > 0.0, y, 0.0)
            return 0
        lax.fori_loop(0, _CH // 16, _ext, 0)

    for t in range(_NT // _CH):
        pltpu.sync_copy(
            x_hbm.at[s, pl.ds(t * _CH, _CH), pl.ds(colbase, _FH)], st2)

        def _g0row(r, _):
            nv = plsc.load_gather(
                nrm, [jnp.full((16,), t * _CH + r, jnp.int32)])
            for q in range(_FH // 16):
                sl = pl.ds(q * 16, 16)
                st1[r, sl] = st2[r, sl] * nv
            return 0
        lax.fori_loop(0, _CH, _g0row, 0)
        pltpu.sync_copy(st1, g_sh.at[nslice(t)])
    plsc.subcore_barrier()

    def _step(step, _):
        for t in range(_NT // _CH):
            pltpu.sync_copy(zb, m_sh.at[nslice(t)])
        plsc.subcore_barrier()

        def _edges(j, _):
            pltpu.sync_copy(src_hbm.at[s, j], sidx)
            pltpu.sync_copy(dst_hbm.at[s, j], didx)
            pltpu.async_copy(g_sh.at[sidx], st1, sem).wait()
            pltpu.sync_copy(st1, m_sh.at[didx], add=True)
            return 0
        lax.fori_loop(0, _NCHUNK, _edges, 0)
        plsc.subcore_barrier()

        keep = jnp.logical_or(step == _K - 1, step == 2 * _K - 1)
        for t in range(_NT // _CH):
            pltpu.sync_copy(m_sh.at[nslice(t)], st1)
            row_sl = pl.ds(t * _CH, _CH)
            col_sl = pl.ds(colbase, _FH)

            @pl.when(step < _K)
            def _():
                pltpu.sync_copy(x_hbm.at[s, row_sl, col_sl], st2)

            @pl.when(step >= _K)
            def _():
                pltpu.sync_copy(out_hbm.at[s, row_sl, col_sl], st2)

            def _crow(r, _):
                nv = plsc.load_gather(
                    nrm, [jnp.full((16,), t * _CH + r, jnp.int32)])
                for q in range(_FH // 16):
                    sl = pl.ds(q * 16, 16)
                    m16 = st1[r, sl]
                    h016 = st2[r, sl]
                    h16 = (1.0 - _ALPHA) * (m16 * nv) + _ALPHA * h016
                    st1[r, sl] = h16 * nv
                    st2[r, sl] = h16
                return 0
            lax.fori_loop(0, _CH, _crow, 0)
            pltpu.sync_copy(st1, g_sh.at[nslice(t)])

            @pl.when(keep)
            def _():
                pltpu.sync_copy(st2, out_hbm.at[s, row_sl, col_sl])
        plsc.subcore_barrier()
        return 0

    lax.fori_loop(0, 2 * _K, _step, 0)


def _propagate(x, edge_index):
    pad_e = _EP - _E
    src = jnp.concatenate(
        [edge_index[0], jnp.full((pad_e,), _N, jnp.int32)]).reshape(
            _NS, _NCHUNK, _CH)
    dst = jnp.concatenate(
        [edge_index[1], jnp.full((pad_e,), _N, jnp.int32)]).reshape(
            _NS, _NCHUNK, _CH)
    xp = jnp.pad(x, ((0, _NP - _N), (0, 0))).reshape(_NS, _NT, _DF)

    mesh = plsc.VectorSubcoreMesh(core_axis_name="c", subcore_axis_name="s",
                                  num_cores=_NC, num_subcores=_NS)
    prop = pl.kernel(
        _prop_body,
        out_type=jax.ShapeDtypeStruct((_NS, _NT, _DF), jnp.float32),
        mesh=mesh,
        scratch_types=[
            pltpu.VMEM_SHARED((_NP, _FH), jnp.float32),
            pltpu.VMEM_SHARED((_NP, _FH), jnp.float32),
            pltpu.VMEM((_CH,), jnp.int32),
            pltpu.VMEM((_CH,), jnp.int32),
            pltpu.VMEM((_CH, _FH), jnp.float32),
            pltpu.VMEM((_CH, _FH), jnp.float32),
            pltpu.VMEM((_CH, _FH), jnp.float32),
            pltpu.VMEM((_NT,), jnp.float32),
            pltpu.SemaphoreType.DMA,
        ],
        compiler_params=pltpu.CompilerParams(use_tc_tiling_on_sc=False,
                                             needs_layout_passes=False),
    )
    return prop(xp, src, dst).reshape(_NP, _DF)


def _mlp_body(x_ref, w1_ref, b1_ref, w2_ref, b2_ref, wa_ref, ba_ref,
              wb_ref, bb_ref, o_ref, h_ref):
    x = x_ref[...]
    h1 = jnp.maximum(
        jnp.dot(x, w1_ref[...], preferred_element_type=jnp.float32)
        + b1_ref[...], 0.0)
    hl = (jnp.dot(h1, w2_ref[...], preferred_element_type=jnp.float32)
          + b2_ref[...])
    h_ref[...] = hl
    for i in range(_L):
        a = jnp.maximum(
            jnp.dot(hl, wa_ref[i], preferred_element_type=jnp.float32)
            + ba_ref[i], 0.0)
        o_ref[i] = (jnp.dot(a, wb_ref[i], preferred_element_type=jnp.float32)
                    + bb_ref[i])


def _mlp_pallas(x, W1, b1, W2, b2, Wa, ba, Wb, bb):
    nblk = _NP // _RB
    grid = (nblk,)
    full = lambda shape: pl.BlockSpec(shape, lambda i: tuple(0 for _ in shape))
    o, h = pl.pallas_call(
        _mlp_body,
        grid=grid,
        in_specs=[
            pl.BlockSpec((_RB, _DF), lambda i: (i, 0)),
            full((_DF, _H)),
            full((1, _H)),
            full((_H, _H)),
            full((1, _H)),
            full((_L, _H, _H // 2)),
            full((_L, 1, _H // 2)),
            full((_L, _H // 2, _CP)),
            full((_L, 1, _CP)),
        ],
        out_specs=[
            pl.BlockSpec((_L, _RB, _CP), lambda i: (0, i, 0)),
            pl.BlockSpec((_RB, _H), lambda i: (i, 0)),
        ],
        out_shape=[
            jax.ShapeDtypeStruct((_L, _NP, _CP), jnp.float32),
            jax.ShapeDtypeStruct((_NP, _H), jnp.float32),
        ],
    )(x, W1, b1.reshape(1, _H), W2, b2.reshape(1, _H),
      Wa, ba.reshape(_L, 1, _H // 2),
      jnp.pad(Wb, ((0, 0), (0, 0), (0, _CP - _C))),
      jnp.pad(bb, ((0, 0), (0, _CP - _C))).reshape(_L, 1, _CP))
    return o, h


def kernel(input_feat, edge_index, W1, b1, W2, b2, Wa, ba, Wb, bb):
    hp = _propagate(input_feat, edge_index)
    o, h_last = _mlp_pallas(hp, W1, b1, W2, b2, Wa, ba, Wb, bb)
    outs = tuple(o[i, :_N, :_C] for i in range(_L))
    return (*outs, h_last[:_N])


# speedup vs baseline: 5.9843x; 1.4148x over previous
import functools

import jax
import jax.numpy as jnp
from jax import lax
from jax.experimental import pallas as pl
from jax.experimental.pallas import tpu as pltpu
from jax.experimental.pallas import tpu_sc as plsc

_N = 10000
_E = 320000
_DF = 128
_H = 256
_C = 40
_L = 4
_K = 10
_ALPHA = 0.1

_NC = 2
_NS = 16
_NP = 10240
_NT = _NP // _NS
_CH = 128
_NCHUNK = 158
_EP = _NS * _NCHUNK * _CH
_EPT = _NCHUNK * _CH
_FH = _DF // _NC
_RB = 1024
_CP = 128


def _rsqrt16(d):
    i = plsc.bitcast(d, jnp.int32)
    i = jnp.int32(0x5F3759DF) - (i >> 1)
    y = plsc.bitcast(i, jnp.float32)
    for _ in range(3):
        y = y * (1.5 - 0.5 * d * y * y)
    return y


def _prop_body(x_hbm, src_hbm, dst_hbm, out_hbm,
               g_sh, m_sh, sidx, didx, st1, st2, zb, stB, nrm, semG, semI):
    c = lax.axis_index("c")
    s = lax.axis_index("s")
    rowbase = s * _NT
    colbase = c * _FH
    zero16 = jnp.zeros((16,), jnp.float32)
    lane = lax.iota(jnp.int32, 16)
    zlane = jnp.zeros((16,), jnp.int32)
    e0 = jnp.where(lane == 0, 1.0, 0.0).astype(jnp.float32)
    nslice = lambda t: pl.ds(rowbase + t * _CH, _CH)

    def _zrow(r, _):
        for q in range(_FH // 16):
            sl = pl.ds(q * 16, 16)
            zb[r, sl] = zero16
            st1[r, sl] = e0 if q == 0 else zero16
        return 0
    lax.fori_loop(0, _CH, _zrow, 0)

    for t in range(_NT // _CH):
        pltpu.sync_copy(zb, m_sh.at[nslice(t)])
    plsc.subcore_barrier()

    def _degc(j, _):
        pltpu.sync_copy(dst_hbm.at[s, j], didx.at[0])
        pltpu.sync_copy(st1, m_sh.at[didx.at[0]], add=True)
        return 0
    lax.fori_loop(0, _NCHUNK, _degc, 0)
    plsc.subcore_barrier()

    for t in range(_NT // _CH):
        pltpu.sync_copy(m_sh.at[nslice(t)], st2)

        def _ext(i, _):
            d = plsc.load_gather(st2, [i * 16 + lane, zlane])
            y = _rsqrt16(jnp.maximum(d, 1.0))
            nrm[pl.ds(t * _CH + i * 16, 16)] = jnp.where(d > 0.0, y, 0.0)
            return 0
        lax.fori_loop(0, _CH // 16, _ext, 0)

    for t in range(_NT // _CH):
        pltpu.sync_copy(
            x_hbm.at[s, pl.ds(t * _CH, _CH), pl.ds(colbase, _FH)], st2)

        def _g0row(r, _):
            nv = plsc.load_gather(
                nrm, [jnp.full((16,), t * _CH + r, jnp.int32)])
            for q in range(_FH // 16):
                sl = pl.ds(q * 16, 16)
                st1[r, sl] = st2[r, sl] * nv
            return 0
        lax.fori_loop(0, _CH, _g0row, 0)
        pltpu.sync_copy(st1, g_sh.at[nslice(t)])
    plsc.subcore_barrier()

    def _step(step, _):
        for t in range(_NT // _CH):
            pltpu.sync_copy(zb, m_sh.at[nslice(t)])
        plsc.subcore_barrier()

        pltpu.sync_copy(src_hbm.at[s, 0], sidx.at[0])
        pltpu.sync_copy(dst_hbm.at[s, 0], didx.at[0])
        pltpu.async_copy(g_sh.at[sidx.at[0]], st1, semG)
        pltpu.async_copy(src_hbm.at[s, 1], sidx.at[1], semI)
        pltpu.async_copy(dst_hbm.at[s, 1], didx.at[1], semI)

        def _pair(jj, _):
            for b, ring, oring in ((0, st1, stB), (1, stB, st1)):
                j = jj * 2 + b
                jnn = jnp.minimum(j + 2, _NCHUNK - 1)
                pltpu.make_async_copy(
                    src_hbm.at[s, 0], sidx.at[1 - b], semI).wait()
                pltpu.make_async_copy(
                    dst_hbm.at[s, 0], didx.at[1 - b], semI).wait()
                pltpu.make_async_copy(
                    x_hbm.at[s, pl.ds(0, _CH), pl.ds(colbase, _FH)],
                    ring, semG).wait()
                pltpu.async_copy(g_sh.at[sidx.at[1 - b]], oring, semG)
                pltpu.sync_copy(ring, m_sh.at[didx.at[b]], add=True)
                pltpu.async_copy(src_hbm.at[s, jnn], sidx.at[b], semI)
                pltpu.async_copy(dst_hbm.at[s, jnn], didx.at[b], semI)
            return 0
        lax.fori_loop(0, _NCHUNK // 2, _pair, 0)
        pltpu.make_async_copy(
            x_hbm.at[s, pl.ds(0, _CH), pl.ds(colbase, _FH)], st1, semG).wait()
        pltpu.make_async_copy(src_hbm.at[s, 0], sidx.at[0], semI).wait()
        pltpu.make_async_copy(dst_hbm.at[s, 0], didx.at[0], semI).wait()
        plsc.subcore_barrier()

        keep = jnp.logical_or(step == _K - 1, step == 2 * _K - 1)
        for t in range(_NT // _CH):
            pltpu.sync_copy(m_sh.at[nslice(t)], st1)
            row_sl = pl.ds(t * _CH, _CH)
            col_sl = pl.ds(colbase, _FH)

            @pl.when(step < _K)
            def _():
                pltpu.sync_copy(x_hbm.at[s, row_sl, col_sl], st2)

            @pl.when(step >= _K)
            def _():
                pltpu.sync_copy(out_hbm.at[s, row_sl, col_sl], st2)

            def _crow(r, _):
                nv = plsc.load_gather(
                    nrm, [jnp.full((16,), t * _CH + r, jnp.int32)])
                for q in range(_FH // 16):
                    sl = pl.ds(q * 16, 16)
                    m16 = st1[r, sl]
                    h016 = st2[r, sl]
                    h16 = (1.0 - _ALPHA) * (m16 * nv) + _ALPHA * h016
                    st1[r, sl] = h16 * nv
                    st2[r, sl] = h16
                return 0
            lax.fori_loop(0, _CH, _crow, 0)
            pltpu.sync_copy(st1, g_sh.at[nslice(t)])

            @pl.when(keep)
            def _():
                pltpu.sync_copy(st2, out_hbm.at[s, row_sl, col_sl])
        plsc.subcore_barrier()
        return 0

    lax.fori_loop(0, 2 * _K, _step, 0)


def _propagate(x, edge_index):
    pad_e = _EP - _E
    src = jnp.concatenate(
        [edge_index[0], jnp.full((pad_e,), _N, jnp.int32)]).reshape(
            _NS, _NCHUNK, _CH)
    dst = jnp.concatenate(
        [edge_index[1], jnp.full((pad_e,), _N, jnp.int32)]).reshape(
            _NS, _NCHUNK, _CH)
    xp = jnp.pad(x, ((0, _NP - _N), (0, 0))).reshape(_NS, _NT, _DF)

    mesh = plsc.VectorSubcoreMesh(core_axis_name="c", subcore_axis_name="s",
                                  num_cores=_NC, num_subcores=_NS)
    prop = pl.kernel(
        _prop_body,
        out_type=jax.ShapeDtypeStruct((_NS, _NT, _DF), jnp.float32),
        mesh=mesh,
        scratch_types=[
            pltpu.VMEM_SHARED((_NP, _FH), jnp.float32),
            pltpu.VMEM_SHARED((_NP, _FH), jnp.float32),
            pltpu.VMEM((2, _CH), jnp.int32),
            pltpu.VMEM((2, _CH), jnp.int32),
            pltpu.VMEM((_CH, _FH), jnp.float32),
            pltpu.VMEM((_CH, _FH), jnp.float32),
            pltpu.VMEM((_CH, _FH), jnp.float32),
            pltpu.VMEM((_CH, _FH), jnp.float32),
            pltpu.VMEM((_NT,), jnp.float32),
            pltpu.SemaphoreType.DMA,
            pltpu.SemaphoreType.DMA,
        ],
        compiler_params=pltpu.CompilerParams(use_tc_tiling_on_sc=False,
                                             needs_layout_passes=False),
    )
    return prop(xp, src, dst).reshape(_NP, _DF)


def _mlp_body(x_ref, w1_ref, b1_ref, w2_ref, b2_ref, wa_ref, ba_ref,
              wb_ref, bb_ref, o_ref, h_ref):
    x = x_ref[...]
    h1 = jnp.maximum(
        jnp.dot(x, w1_ref[...], preferred_element_type=jnp.float32)
        + b1_ref[...], 0.0)
    hl = (jnp.dot(h1, w2_ref[...], preferred_element_type=jnp.float32)
          + b2_ref[...])
    h_ref[...] = hl
    for i in range(_L):
        a = jnp.maximum(
            jnp.dot(hl, wa_ref[i], preferred_element_type=jnp.float32)
            + ba_ref[i], 0.0)
        o_ref[i] = (jnp.dot(a, wb_ref[i], preferred_element_type=jnp.float32)
                    + bb_ref[i])


def _mlp_pallas(x, W1, b1, W2, b2, Wa, ba, Wb, bb):
    nblk = _NP // _RB
    grid = (nblk,)
    full = lambda shape: pl.BlockSpec(shape, lambda i: tuple(0 for _ in shape))
    o, h = pl.pallas_call(
        _mlp_body,
        grid=grid,
        in_specs=[
            pl.BlockSpec((_RB, _DF), lambda i: (i, 0)),
            full((_DF, _H)),
            full((1, _H)),
            full((_H, _H)),
            full((1, _H)),
            full((_L, _H, _H // 2)),
            full((_L, 1, _H // 2)),
            full((_L, _H // 2, _CP)),
            full((_L, 1, _CP)),
        ],
        out_specs=[
            pl.BlockSpec((_L, _RB, _CP), lambda i: (0, i, 0)),
            pl.BlockSpec((_RB, _H), lambda i: (i, 0)),
        ],
        out_shape=[
            jax.ShapeDtypeStruct((_L, _NP, _CP), jnp.float32),
            jax.ShapeDtypeStruct((_NP, _H), jnp.float32),
        ],
    )(x, W1, b1.reshape(1, _H), W2, b2.reshape(1, _H),
      Wa, ba.reshape(_L, 1, _H // 2),
      jnp.pad(Wb, ((0, 0), (0, 0), (0, _CP - _C))),
      jnp.pad(bb, ((0, 0), (0, _CP - _C))).reshape(_L, 1, _CP))
    return o, h


def kernel(input_feat, edge_index, W1, b1, W2, b2, Wa, ba, Wb, bb):
    hp = _propagate(input_feat, edge_index)
    o, h_last = _mlp_pallas(hp, W1, b1, W2, b2, Wa, ba, Wb, bb)
    outs = tuple(o[i, :_N, :_C] for i in range(_L))
    return (*outs, h_last[:_N])


# speedup vs baseline: 8.5835x; 1.4343x over previous
import functools

import jax
import jax.numpy as jnp
from jax import lax
from jax.experimental import pallas as pl
from jax.experimental.pallas import tpu as pltpu
from jax.experimental.pallas import tpu_sc as plsc

_N = 10000
_E = 320000
_DF = 128
_H = 256
_C = 40
_L = 4
_K = 10
_ALPHA = 0.1

_NC = 2
_NS = 16
_NP = 10240
_NT = _NP // _NS
_CH = 128
_NCHUNK = 160
_EP = _NS * _NCHUNK * _CH
_FH = _DF // _NC
_RB = 1024
_CP = 128


def _rsqrt16(d):
    i = plsc.bitcast(d, jnp.int32)
    i = jnp.int32(0x5F3759DF) - (i >> 1)
    y = plsc.bitcast(i, jnp.float32)
    for _ in range(3):
        y = y * (1.5 - 0.5 * d * y * y)
    return y


def _prop_body(x_hbm, e_hbm, out_hbm,
               g_sh, m_sh, idx8, ring4, zb, nrm, semG, semI):
    c = lax.axis_index("c")
    s = lax.axis_index("s")
    rowbase = s * _NT
    colbase = c * _FH
    zero16 = jnp.zeros((16,), jnp.float32)
    lane = lax.iota(jnp.int32, 16)
    zlane = jnp.zeros((16,), jnp.int32)
    one16i = jnp.full((16,), 1, jnp.int32)
    e0 = jnp.where(lane == 0, 1.0, 0.0).astype(jnp.float32)
    nslice = lambda t: pl.ds(rowbase + t * _CH, _CH)
    xchunk = lambda t: x_hbm.at[s, pl.ds(t * _CH, _CH), pl.ds(colbase, _FH)]
    ochunk = lambda t: out_hbm.at[s, pl.ds(t * _CH, _CH), pl.ds(colbase, _FH)]
    gdummy = x_hbm.at[s, pl.ds(0, _CH), pl.ds(colbase, _FH)]
    idummy = e_hbm.at[s, 0]

    def _zrow(r, _):
        for q in range(_FH // 16):
            sl = pl.ds(q * 16, 16)
            zb[r, sl] = zero16
            ring4[0, r, sl] = e0 if q == 0 else zero16
        return 0
    lax.fori_loop(0, _CH, _zrow, 0)

    for t in range(_NT // _CH):
        pltpu.sync_copy(zb, m_sh.at[nslice(t)])
    plsc.subcore_barrier()

    def _degc(j, _):
        pltpu.sync_copy(e_hbm.at[s, j], idx8.at[0])
        pltpu.sync_copy(ring4.at[0], m_sh.at[idx8.at[0, 1]], add=True)
        return 0
    lax.fori_loop(0, _NCHUNK, _degc, 0)
    plsc.subcore_barrier()

    for t in range(_NT // _CH):
        pltpu.sync_copy(m_sh.at[nslice(t)], ring4.at[1])

        def _ext(i, _):
            d = plsc.load_gather(ring4, [one16i, i * 16 + lane, zlane])
            y = _rsqrt16(jnp.maximum(d, 1.0))
            nrm[pl.ds(t * _CH + i * 16, 16)] = jnp.where(d > 0.0, y, 0.0)
            return 0
        lax.fori_loop(0, _CH // 16, _ext, 0)

    for t in range(_NT // _CH):
        pltpu.sync_copy(xchunk(t), ring4.at[1])

        def _g0row(r, _):
            nv = plsc.load_gather(
                nrm, [jnp.full((16,), t * _CH + r, jnp.int32)])
            for q in range(_FH // 16):
                sl = pl.ds(q * 16, 16)
                ring4[0, r, sl] = ring4[1, r, sl] * nv
            return 0
        lax.fori_loop(0, _CH, _g0row, 0)
        pltpu.sync_copy(ring4.at[0], g_sh.at[nslice(t)])
    plsc.subcore_barrier()

    def _step(step, _):
        for t in range(_NT // _CH):
            pltpu.sync_copy(zb, m_sh.at[nslice(t)])
        plsc.subcore_barrier()

        pltpu.sync_copy(e_hbm.at[s, 0], idx8.at[0])
        pltpu.sync_copy(e_hbm.at[s, 1], idx8.at[1])
        pltpu.async_copy(g_sh.at[idx8.at[0, 0]], ring4.at[0], semG)
        pltpu.async_copy(g_sh.at[idx8.at[1, 0]], ring4.at[1], semG)
        for p in range(2, 6):
            pltpu.async_copy(e_hbm.at[s, p], idx8.at[p], semI)

        def _edge(j, _):
            b4 = lax.rem(j, 4)
            b8 = lax.rem(j, 8)
            pltpu.make_async_copy(gdummy, ring4.at[0], semG).wait()
            pltpu.sync_copy(ring4.at[b4], m_sh.at[idx8.at[b8, 1]], add=True)

            pltpu.make_async_copy(idummy, idx8.at[0], semI).wait()
            pltpu.async_copy(g_sh.at[idx8.at[lax.rem(j + 2, 8), 0]],
                             ring4.at[lax.rem(j + 2, 4)], semG)
            pltpu.async_copy(e_hbm.at[s, jnp.minimum(j + 6, _NCHUNK - 1)],
                             idx8.at[lax.rem(j + 6, 8)], semI)
            return 0
        lax.fori_loop(0, _NCHUNK, _edge, 0)
        for _i in range(2):
            pltpu.make_async_copy(gdummy, ring4.at[0], semG).wait()
        for _i in range(4):
            pltpu.make_async_copy(idummy, idx8.at[0], semI).wait()
        plsc.subcore_barrier()

        keep = jnp.logical_or(step == _K - 1, step == 2 * _K - 1)
        for t in range(_NT // _CH):
            pltpu.sync_copy(m_sh.at[nslice(t)], ring4.at[0])

            @pl.when(step < _K)
            def _():
                pltpu.sync_copy(xchunk(t), ring4.at[1])

            @pl.when(step >= _K)
            def _():
                pltpu.sync_copy(ochunk(t), ring4.at[1])

            def _crow(r, _):
                nv = plsc.load_gather(
                    nrm, [jnp.full((16,), t * _CH + r, jnp.int32)])
                for q in range(_FH // 16):
                    sl = pl.ds(q * 16, 16)
                    m16 = ring4[0, r, sl]
                    h016 = ring4[1, r, sl]
                    h16 = (1.0 - _ALPHA) * (m16 * nv) + _ALPHA * h016
                    ring4[2, r, sl] = h16 * nv
                    ring4[3, r, sl] = h16
                return 0
            lax.fori_loop(0, _CH, _crow, 0)
            pltpu.sync_copy(ring4.at[2], g_sh.at[nslice(t)])

            @pl.when(keep)
            def _():
                pltpu.sync_copy(ring4.at[3], ochunk(t))
        plsc.subcore_barrier()
        return 0

    lax.fori_loop(0, 2 * _K, _step, 0)


def _propagate(x, edge_index):
    pad_e = _EP - _E
    src = jnp.concatenate(
        [edge_index[0], jnp.full((pad_e,), _N, jnp.int32)]).reshape(
            _NS, _NCHUNK, _CH)
    dst = jnp.concatenate(
        [edge_index[1], jnp.full((pad_e,), _N, jnp.int32)]).reshape(
            _NS, _NCHUNK, _CH)
    e = jnp.stack([src, dst], axis=2)
    xp = jnp.pad(x, ((0, _NP - _N), (0, 0))).reshape(_NS, _NT, _DF)

    mesh = plsc.VectorSubcoreMesh(core_axis_name="c", subcore_axis_name="s",
                                  num_cores=_NC, num_subcores=_NS)
    prop = pl.kernel(
        _prop_body,
        out_type=jax.ShapeDtypeStruct((_NS, _NT, _DF), jnp.float32),
        mesh=mesh,
        scratch_types=[
            pltpu.VMEM_SHARED((_NP, _FH), jnp.float32),
            pltpu.VMEM_SHARED((_NP, _FH), jnp.float32),
            pltpu.VMEM((8, 2, _CH), jnp.int32),
            pltpu.VMEM((4, _CH, _FH), jnp.float32),
            pltpu.VMEM((_CH, _FH), jnp.float32),
            pltpu.VMEM((_NT,), jnp.float32),
            pltpu.SemaphoreType.DMA,
            pltpu.SemaphoreType.DMA,
        ],
        compiler_params=pltpu.CompilerParams(use_tc_tiling_on_sc=False,
                                             needs_layout_passes=False),
    )
    return prop(xp, e).reshape(_NP, _DF)


def _mlp_body(x_ref, w1_ref, b1_ref, w2_ref, b2_ref, wa_ref, ba_ref,
              wb_ref, bb_ref, o_ref, h_ref):
    x = x_ref[...]
    h1 = jnp.maximum(
        jnp.dot(x, w1_ref[...], preferred_element_type=jnp.float32)
        + b1_ref[...], 0.0)
    hl = (jnp.dot(h1, w2_ref[...], preferred_element_type=jnp.float32)
          + b2_ref[...])
    h_ref[...] = hl
    for i in range(_L):
        a = jnp.maximum(
            jnp.dot(hl, wa_ref[i], preferred_element_type=jnp.float32)
            + ba_ref[i], 0.0)
        o_ref[i] = (jnp.dot(a, wb_ref[i], preferred_element_type=jnp.float32)
                    + bb_ref[i])


def _mlp_pallas(x, W1, b1, W2, b2, Wa, ba, Wb, bb):
    nblk = _NP // _RB
    grid = (nblk,)
    full = lambda shape: pl.BlockSpec(shape, lambda i: tuple(0 for _ in shape))
    o, h = pl.pallas_call(
        _mlp_body,
        grid=grid,
        in_specs=[
            pl.BlockSpec((_RB, _DF), lambda i: (i, 0)),
            full((_DF, _H)),
            full((1, _H)),
            full((_H, _H)),
            full((1, _H)),
            full((_L, _H, _H // 2)),
            full((_L, 1, _H // 2)),
            full((_L, _H // 2, _CP)),
            full((_L, 1, _CP)),
        ],
        out_specs=[
            pl.BlockSpec((_L, _RB, _CP), lambda i: (0, i, 0)),
            pl.BlockSpec((_RB, _H), lambda i: (i, 0)),
        ],
        out_shape=[
            jax.ShapeDtypeStruct((_L, _NP, _CP), jnp.float32),
            jax.ShapeDtypeStruct((_NP, _H), jnp.float32),
        ],
    )(x, W1, b1.reshape(1, _H), W2, b2.reshape(1, _H),
      Wa, ba.reshape(_L, 1, _H // 2),
      jnp.pad(Wb, ((0, 0), (0, 0), (0, _CP - _C))),
      jnp.pad(bb, ((0, 0), (0, _CP - _C))).reshape(_L, 1, _CP))
    return o, h


def kernel(input_feat, edge_index, W1, b1, W2, b2, Wa, ba, Wb, bb):
    hp = _propagate(input_feat, edge_index)
    o, h_last = _mlp_pallas(hp, W1, b1, W2, b2, Wa, ba, Wb, bb)
    outs = tuple(o[i, :_N, :_C] for i in range(_L))
    return (*outs, h_last[:_N])


# speedup vs baseline: 8.7350x; 1.0177x over previous
import functools

import jax
import jax.numpy as jnp
from jax import lax
from jax.experimental import pallas as pl
from jax.experimental.pallas import tpu as pltpu
from jax.experimental.pallas import tpu_sc as plsc

_N = 10000
_E = 320000
_DF = 128
_H = 256
_C = 40
_L = 4
_K = 10
_ALPHA = 0.1

_NC = 2
_NS = 16
_NP = 10240
_NT = _NP // _NS
_CH = 128
_NCHUNK = 160
_EP = _NS * _NCHUNK * _CH
_FH = _DF // _NC
_RB = 1024
_CP = 128


def _rsqrt16(d):
    i = plsc.bitcast(d, jnp.int32)
    i = jnp.int32(0x5F3759DF) - (i >> 1)
    y = plsc.bitcast(i, jnp.float32)
    for _ in range(3):
        y = y * (1.5 - 0.5 * d * y * y)
    return y


def _prop_body(x_hbm, e_hbm, out_hbm,
               g_sh, m_sh, idx8, ring4, zb, nrm,
               semG0, semG1, semI):
    c = lax.axis_index("c")
    s = lax.axis_index("s")
    rowbase = s * _NT
    colbase = c * _FH
    zero16 = jnp.zeros((16,), jnp.float32)
    lane = lax.iota(jnp.int32, 16)
    zlane = jnp.zeros((16,), jnp.int32)
    one16i = jnp.full((16,), 1, jnp.int32)
    e0 = jnp.where(lane == 0, 1.0, 0.0).astype(jnp.float32)
    nslice = lambda t: pl.ds(rowbase + t * _CH, _CH)
    xchunk = lambda t: x_hbm.at[s, pl.ds(t * _CH, _CH), pl.ds(colbase, _FH)]
    ochunk = lambda t: out_hbm.at[s, pl.ds(t * _CH, _CH), pl.ds(colbase, _FH)]
    gdummy = x_hbm.at[s, pl.ds(0, _CH), pl.ds(colbase, _FH)]
    idummy = e_hbm.at[s, 0]

    def _zrow(r, _):
        for q in range(_FH // 16):
            sl = pl.ds(q * 16, 16)
            zb[r, sl] = zero16
            ring4[0, r, sl] = e0 if q == 0 else zero16
        return 0
    lax.fori_loop(0, _CH, _zrow, 0)

    for t in range(_NT // _CH):
        pltpu.sync_copy(zb, m_sh.at[nslice(t)])
    plsc.subcore_barrier()

    def _degc(j, _):
        pltpu.sync_copy(e_hbm.at[s, j], idx8.at[0])
        pltpu.sync_copy(ring4.at[0], m_sh.at[idx8.at[0, 1]], add=True)
        return 0
    lax.fori_loop(0, _NCHUNK, _degc, 0)
    plsc.subcore_barrier()

    for t in range(_NT // _CH):
        pltpu.sync_copy(m_sh.at[nslice(t)], ring4.at[1])

        def _ext(i, _):
            d = plsc.load_gather(ring4, [one16i, i * 16 + lane, zlane])
            y = _rsqrt16(jnp.maximum(d, 1.0))
            nrm[pl.ds(t * _CH + i * 16, 16)] = jnp.where(d > 0.0, y, 0.0)
            return 0
        lax.fori_loop(0, _CH // 16, _ext, 0)

    for t in range(_NT // _CH):
        pltpu.sync_copy(xchunk(t), ring4.at[1])

        def _g0row(r, _):
            nv = plsc.load_gather(
                nrm, [jnp.full((16,), t * _CH + r, jnp.int32)])
            for q in range(_FH // 16):
                sl = pl.ds(q * 16, 16)
                ring4[0, r, sl] = ring4[1, r, sl] * nv
            return 0
        lax.fori_loop(0, _CH, _g0row, 0)
        pltpu.sync_copy(ring4.at[0], g_sh.at[nslice(t)])
    plsc.subcore_barrier()

    def _step(step, _):
        for t in range(_NT // _CH):
            pltpu.sync_copy(zb, m_sh.at[nslice(t)])
        plsc.subcore_barrier()

        pltpu.sync_copy(e_hbm.at[s, 0], idx8.at[0])
        pltpu.sync_copy(e_hbm.at[s, 1], idx8.at[1])
        pltpu.async_copy(g_sh.at[idx8.at[0, 0]], ring4.at[0], semG0)
        pltpu.async_copy(g_sh.at[idx8.at[1, 0]], ring4.at[1], semG1)
        for p in range(2, 6):
            pltpu.async_copy(e_hbm.at[s, p], idx8.at[p], semI)

        def _pair(jj, _):
            for b, semG in ((0, semG0), (1, semG1)):
                j = jj * 2 + b
                b4 = lax.rem(j, 4)
                b8 = lax.rem(j, 8)
                pltpu.make_async_copy(gdummy, ring4.at[0], semG).wait()
                pltpu.make_async_copy(idummy, idx8.at[0], semI).wait()
                pltpu.async_copy(g_sh.at[idx8.at[lax.rem(j + 2, 8), 0]],
                                 ring4.at[lax.rem(j + 2, 4)], semG)
                pltpu.async_copy(
                    e_hbm.at[s, jnp.minimum(j + 6, _NCHUNK - 1)],
                    idx8.at[lax.rem(j + 6, 8)], semI)
                pltpu.sync_copy(ring4.at[b4], m_sh.at[idx8.at[b8, 1]],
                                add=True)
            return 0
        lax.fori_loop(0, _NCHUNK // 2, _pair, 0)
        for sg in (semG0, semG1):
            pltpu.make_async_copy(gdummy, ring4.at[0], sg).wait()
        for _i in range(4):
            pltpu.make_async_copy(idummy, idx8.at[0], semI).wait()
        plsc.subcore_barrier()

        keep = jnp.logical_or(step == _K - 1, step == 2 * _K - 1)
        for t in range(_NT // _CH):
            pltpu.sync_copy(m_sh.at[nslice(t)], ring4.at[0])

            @pl.when(step < _K)
            def _():
                pltpu.sync_copy(xchunk(t), ring4.at[1])

            @pl.when(step >= _K)
            def _():
                pltpu.sync_copy(ochunk(t), ring4.at[1])

            def _crow(r, _):
                nv = plsc.load_gather(
                    nrm, [jnp.full((16,), t * _CH + r, jnp.int32)])
                for q in range(_FH // 16):
                    sl = pl.ds(q * 16, 16)
                    m16 = ring4[0, r, sl]
                    h016 = ring4[1, r, sl]
                    h16 = (1.0 - _ALPHA) * (m16 * nv) + _ALPHA * h016
                    ring4[2, r, sl] = h16 * nv
                    ring4[3, r, sl] = h16
                return 0
            lax.fori_loop(0, _CH, _crow, 0)
            pltpu.sync_copy(ring4.at[2], g_sh.at[nslice(t)])

            @pl.when(keep)
            def _():
                pltpu.sync_copy(ring4.at[3], ochunk(t))
        plsc.subcore_barrier()
        return 0

    lax.fori_loop(0, 2 * _K, _step, 0)


def _propagate(x, edge_index):
    pad_e = _EP - _E
    src = jnp.concatenate(
        [edge_index[0], jnp.full((pad_e,), _N, jnp.int32)]).reshape(
            _NS, _NCHUNK, _CH)
    dst = jnp.concatenate(
        [edge_index[1], jnp.full((pad_e,), _N, jnp.int32)]).reshape(
            _NS, _NCHUNK, _CH)
    e = jnp.stack([src, dst], axis=2)
    xp = jnp.pad(x, ((0, _NP - _N), (0, 0))).reshape(_NS, _NT, _DF)

    mesh = plsc.VectorSubcoreMesh(core_axis_name="c", subcore_axis_name="s",
                                  num_cores=_NC, num_subcores=_NS)
    prop = pl.kernel(
        _prop_body,
        out_type=jax.ShapeDtypeStruct((_NS, _NT, _DF), jnp.float32),
        mesh=mesh,
        scratch_types=[
            pltpu.VMEM_SHARED((_NP, _FH), jnp.float32),
            pltpu.VMEM_SHARED((_NP, _FH), jnp.float32),
            pltpu.VMEM((8, 2, _CH), jnp.int32),
            pltpu.VMEM((4, _CH, _FH), jnp.float32),
            pltpu.VMEM((_CH, _FH), jnp.float32),
            pltpu.VMEM((_NT,), jnp.float32),
            pltpu.SemaphoreType.DMA,
            pltpu.SemaphoreType.DMA,
            pltpu.SemaphoreType.DMA,
        ],
        compiler_params=pltpu.CompilerParams(use_tc_tiling_on_sc=False,
                                             needs_layout_passes=False),
    )
    return prop(xp, e).reshape(_NP, _DF)


def _mlp_body(x_ref, w1_ref, b1_ref, w2_ref, b2_ref, wa_ref, ba_ref,
              wb_ref, bb_ref, o_ref, h_ref):
    x = x_ref[...]
    h1 = jnp.maximum(
        jnp.dot(x, w1_ref[...], preferred_element_type=jnp.float32)
        + b1_ref[...], 0.0)
    hl = (jnp.dot(h1, w2_ref[...], preferred_element_type=jnp.float32)
          + b2_ref[...])
    h_ref[...] = hl
    for i in range(_L):
        a = jnp.maximum(
            jnp.dot(hl, wa_ref[i], preferred_element_type=jnp.float32)
            + ba_ref[i], 0.0)
        o_ref[i] = (jnp.dot(a, wb_ref[i], preferred_element_type=jnp.float32)
                    + bb_ref[i])


def _mlp_pallas(x, W1, b1, W2, b2, Wa, ba, Wb, bb):
    nblk = _NP // _RB
    grid = (nblk,)
    full = lambda shape: pl.BlockSpec(shape, lambda i: tuple(0 for _ in shape))
    o, h = pl.pallas_call(
        _mlp_body,
        grid=grid,
        in_specs=[
            pl.BlockSpec((_RB, _DF), lambda i: (i, 0)),
            full((_DF, _H)),
            full((1, _H)),
            full((_H, _H)),
            full((1, _H)),
            full((_L, _H, _H // 2)),
            full((_L, 1, _H // 2)),
            full((_L, _H // 2, _CP)),
            full((_L, 1, _CP)),
        ],
        out_specs=[
            pl.BlockSpec((_L, _RB, _CP), lambda i: (0, i, 0)),
            pl.BlockSpec((_RB, _H), lambda i: (i, 0)),
        ],
        out_shape=[
            jax.ShapeDtypeStruct((_L, _NP, _CP), jnp.float32),
            jax.ShapeDtypeStruct((_NP, _H), jnp.float32),
        ],
    )(x, W1, b1.reshape(1, _H), W2, b2.reshape(1, _H),
      Wa, ba.reshape(_L, 1, _H // 2),
      jnp.pad(Wb, ((0, 0), (0, 0), (0, _CP - _C))),
      jnp.pad(bb, ((0, 0), (0, _CP - _C))).reshape(_L, 1, _CP))
    return o, h


def kernel(input_feat, edge_index, W1, b1, W2, b2, Wa, ba, Wb, bb):
    hp = _propagate(input_feat, edge_index)
    o, h_last = _mlp_pallas(hp, W1, b1, W2, b2, Wa, ba, Wb, bb)
    outs = tuple(o[i, :_N, :_C] for i in range(_L))
    return (*outs, h_last[:_N])


# speedup vs baseline: 8.7748x; 1.0046x over previous
import functools

import jax
import jax.numpy as jnp
from jax import lax
from jax.experimental import pallas as pl
from jax.experimental.pallas import tpu as pltpu
from jax.experimental.pallas import tpu_sc as plsc

_N = 10000
_E = 320000
_DF = 128
_H = 256
_C = 40
_L = 4
_K = 10
_ALPHA = 0.1

_NC = 2
_NS = 16
_NP = 10240
_NT = _NP // _NS
_CH = 128
_NCHUNK = 160
_EP = _NS * _NCHUNK * _CH
_FH = _DF // _NC
_RB = 1024
_CP = 128


def _rsqrt16(d):
    i = plsc.bitcast(d, jnp.int32)
    i = jnp.int32(0x5F3759DF) - (i >> 1)
    y = plsc.bitcast(i, jnp.float32)
    for _ in range(3):
        y = y * (1.5 - 0.5 * d * y * y)
    return y


def _prop_body(x_hbm, e_hbm, out_hbm,
               g_sh, m_sh, idx8, ring4, zb, nrm,
               semG0, semG1, semI):
    c = lax.axis_index("c")
    s = lax.axis_index("s")
    rowbase = s * _NT
    colbase = c * _FH
    zero16 = jnp.zeros((16,), jnp.float32)
    lane = lax.iota(jnp.int32, 16)
    zlane = jnp.zeros((16,), jnp.int32)
    one16i = jnp.full((16,), 1, jnp.int32)
    e0 = jnp.where(lane == 0, 1.0, 0.0).astype(jnp.float32)
    nslice = lambda t: pl.ds(rowbase + t * _CH, _CH)
    xchunk = lambda t: x_hbm.at[s, pl.ds(t * _CH, _CH), pl.ds(colbase, _FH)]
    ochunk = lambda t: out_hbm.at[s, pl.ds(t * _CH, _CH), pl.ds(colbase, _FH)]
    gdummy = x_hbm.at[s, pl.ds(0, _CH), pl.ds(colbase, _FH)]
    idummy = e_hbm.at[s, 0]

    def _zrow(r, _):
        for q in range(_FH // 16):
            sl = pl.ds(q * 16, 16)
            zb[r, sl] = zero16
            ring4[0, r, sl] = e0 if q == 0 else zero16
        return 0
    lax.fori_loop(0, _CH, _zrow, 0)

    for t in range(_NT // _CH):
        pltpu.sync_copy(zb, m_sh.at[nslice(t)])
    plsc.subcore_barrier()

    def _degc(j, _):
        pltpu.sync_copy(e_hbm.at[s, j], idx8.at[0])
        pltpu.sync_copy(ring4.at[0], m_sh.at[idx8.at[0, 1]], add=True)
        return 0
    lax.fori_loop(0, _NCHUNK, _degc, 0)
    plsc.subcore_barrier()

    for t in range(_NT // _CH):
        pltpu.sync_copy(m_sh.at[nslice(t)], ring4.at[1])

        def _ext(i, _):
            d = plsc.load_gather(ring4, [one16i, i * 16 + lane, zlane])
            y = _rsqrt16(jnp.maximum(d, 1.0))
            nrm[pl.ds(t * _CH + i * 16, 16)] = jnp.where(d > 0.0, y, 0.0)
            return 0
        lax.fori_loop(0, _CH // 16, _ext, 0)

    for t in range(_NT // _CH):
        pltpu.sync_copy(xchunk(t), ring4.at[1])

        def _g0row(r, _):
            nv = plsc.load_gather(
                nrm, [jnp.full((16,), t * _CH + r, jnp.int32)])
            for q in range(_FH // 16):
                sl = pl.ds(q * 16, 16)
                ring4[0, r, sl] = ring4[1, r, sl] * nv
            return 0
        lax.fori_loop(0, _CH, _g0row, 0)
        pltpu.sync_copy(ring4.at[0], g_sh.at[nslice(t)])
    plsc.subcore_barrier()

    def _step(step, _):
        for t in range(_NT // _CH):
            pltpu.sync_copy(zb, m_sh.at[nslice(t)])
        plsc.subcore_barrier()

        pltpu.sync_copy(e_hbm.at[s, 0], idx8.at[0])
        pltpu.sync_copy(e_hbm.at[s, 1], idx8.at[1])
        pltpu.async_copy(g_sh.at[idx8.at[0, 0]], ring4.at[0], semG0)
        pltpu.async_copy(g_sh.at[idx8.at[1, 0]], ring4.at[1], semG1)
        for p in range(2, 6):
            pltpu.async_copy(e_hbm.at[s, p], idx8.at[p], semI)

        def _pair(jj, _):
            for b, semG in ((0, semG0), (1, semG1)):
                j = jj * 2 + b
                b4 = lax.rem(j, 4)
                b8 = lax.rem(j, 8)
                pltpu.make_async_copy(gdummy, ring4.at[0], semG).wait()
                pltpu.make_async_copy(idummy, idx8.at[0], semI).wait()
                pltpu.async_copy(g_sh.at[idx8.at[lax.rem(j + 2, 8), 0]],
                                 ring4.at[lax.rem(j + 2, 4)], semG)
                pltpu.async_copy(
                    e_hbm.at[s, jnp.minimum(j + 6, _NCHUNK - 1)],
                    idx8.at[lax.rem(j + 6, 8)], semI)
                pltpu.sync_copy(ring4.at[b4], m_sh.at[idx8.at[b8, 1]],
                                add=True)
            return 0
        lax.fori_loop(0, _NCHUNK // 2, _pair, 0)
        for sg in (semG0, semG1):
            pltpu.make_async_copy(gdummy, ring4.at[0], sg).wait()
        for _i in range(4):
            pltpu.make_async_copy(idummy, idx8.at[0], semI).wait()
        plsc.subcore_barrier()

        keep = jnp.logical_or(step == _K - 1, step == 2 * _K - 1)
        for t in range(_NT // _CH):
            pltpu.sync_copy(m_sh.at[nslice(t)], ring4.at[0])

            @pl.when(step < _K)
            def _():
                pltpu.sync_copy(xchunk(t), ring4.at[1])

            @pl.when(step >= _K)
            def _():
                pltpu.sync_copy(ochunk(t), ring4.at[1])

            def _crow(r, _):
                nv = plsc.load_gather(
                    nrm, [jnp.full((16,), t * _CH + r, jnp.int32)])
                for q in range(_FH // 16):
                    sl = pl.ds(q * 16, 16)
                    m16 = ring4[0, r, sl]
                    h016 = ring4[1, r, sl]
                    h16 = (1.0 - _ALPHA) * (m16 * nv) + _ALPHA * h016
                    ring4[2, r, sl] = h16 * nv
                return 0

            def _crow_keep(r, _):
                nv = plsc.load_gather(
                    nrm, [jnp.full((16,), t * _CH + r, jnp.int32)])
                for q in range(_FH // 16):
                    sl = pl.ds(q * 16, 16)
                    m16 = ring4[0, r, sl]
                    h016 = ring4[1, r, sl]
                    h16 = (1.0 - _ALPHA) * (m16 * nv) + _ALPHA * h016
                    ring4[2, r, sl] = h16 * nv
                    ring4[3, r, sl] = h16
                return 0

            @pl.when(jnp.logical_not(keep))
            def _():
                lax.fori_loop(0, _CH, _crow, 0)

            @pl.when(keep)
            def _():
                lax.fori_loop(0, _CH, _crow_keep, 0)

            pltpu.sync_copy(ring4.at[2], g_sh.at[nslice(t)])

            @pl.when(keep)
            def _():
                pltpu.sync_copy(ring4.at[3], ochunk(t))
        plsc.subcore_barrier()
        return 0

    lax.fori_loop(0, 2 * _K, _step, 0)


def _propagate(x, edge_index):
    pad_e = _EP - _E
    src = jnp.concatenate(
        [edge_index[0], jnp.full((pad_e,), _N, jnp.int32)]).reshape(
            _NS, _NCHUNK, _CH)
    dst = jnp.concatenate(
        [edge_index[1], jnp.full((pad_e,), _N, jnp.int32)]).reshape(
            _NS, _NCHUNK, _CH)
    e = jnp.stack([src, dst], axis=2)
    xp = jnp.pad(x, ((0, _NP - _N), (0, 0))).reshape(_NS, _NT, _DF)

    mesh = plsc.VectorSubcoreMesh(core_axis_name="c", subcore_axis_name="s",
                                  num_cores=_NC, num_subcores=_NS)
    prop = pl.kernel(
        _prop_body,
        out_type=jax.ShapeDtypeStruct((_NS, _NT, _DF), jnp.float32),
        mesh=mesh,
        scratch_types=[
            pltpu.VMEM_SHARED((_NP, _FH), jnp.float32),
            pltpu.VMEM_SHARED((_NP, _FH), jnp.float32),
            pltpu.VMEM((8, 2, _CH), jnp.int32),
            pltpu.VMEM((4, _CH, _FH), jnp.float32),
            pltpu.VMEM((_CH, _FH), jnp.float32),
            pltpu.VMEM((_NT,), jnp.float32),
            pltpu.SemaphoreType.DMA,
            pltpu.SemaphoreType.DMA,
            pltpu.SemaphoreType.DMA,
        ],
        compiler_params=pltpu.CompilerParams(use_tc_tiling_on_sc=False,
                                             needs_layout_passes=False),
    )
    return prop(xp, e).reshape(_NP, _DF)


def _mlp_body(x_ref, w1_ref, b1_ref, w2_ref, b2_ref, wa_ref, ba_ref,
              wb_ref, bb_ref, o_ref, h_ref):
    x = x_ref[...]
    h1 = jnp.maximum(
        jnp.dot(x, w1_ref[...], preferred_element_type=jnp.float32)
        + b1_ref[...], 0.0)
    hl = (jnp.dot(h1, w2_ref[...], preferred_element_type=jnp.float32)
          + b2_ref[...])
    h_ref[...] = hl
    for i in range(_L):
        a = jnp.maximum(
            jnp.dot(hl, wa_ref[i], preferred_element_type=jnp.float32)
            + ba_ref[i], 0.0)
        o_ref[i] = (jnp.dot(a, wb_ref[i], preferred_element_type=jnp.float32)
                    + bb_ref[i])


def _mlp_pallas(x, W1, b1, W2, b2, Wa, ba, Wb, bb):
    nblk = _NP // _RB
    grid = (nblk,)
    full = lambda shape: pl.BlockSpec(shape, lambda i: tuple(0 for _ in shape))
    o, h = pl.pallas_call(
        _mlp_body,
        grid=grid,
        in_specs=[
            pl.BlockSpec((_RB, _DF), lambda i: (i, 0)),
            full((_DF, _H)),
            full((1, _H)),
            full((_H, _H)),
            full((1, _H)),
            full((_L, _H, _H // 2)),
            full((_L, 1, _H // 2)),
            full((_L, _H // 2, _CP)),
            full((_L, 1, _CP)),
        ],
        out_specs=[
            pl.BlockSpec((_L, _RB, _CP), lambda i: (0, i, 0)),
            pl.BlockSpec((_RB, _H), lambda i: (i, 0)),
        ],
        out_shape=[
            jax.ShapeDtypeStruct((_L, _NP, _CP), jnp.float32),
            jax.ShapeDtypeStruct((_NP, _H), jnp.float32),
        ],
    )(x, W1, b1.reshape(1, _H), W2, b2.reshape(1, _H),
      Wa, ba.reshape(_L, 1, _H // 2),
      jnp.pad(Wb, ((0, 0), (0, 0), (0, _CP - _C))),
      jnp.pad(bb, ((0, 0), (0, _CP - _C))).reshape(_L, 1, _CP))
    return o, h


def kernel(input_feat, edge_index, W1, b1, W2, b2, Wa, ba, Wb, bb):
    hp = _propagate(input_feat, edge_index)
    o, h_last = _mlp_pallas(hp, W1, b1, W2, b2, Wa, ba, Wb, bb)
    outs = tuple(o[i, :_N, :_C] for i in range(_L))
    return (*outs, h_last[:_N])
